# baseline (speedup 1.0000x reference)
import functools

import jax
import jax.numpy as jnp
from jax.experimental import pallas as pl
from jax.experimental.pallas import tpu as pltpu

_N = 10000
_E = 320000
_D_IN = 128
_D_HID = 256
_D_OUT = 128


def _dense1_body(x_ref, W1_ref, b1_ref, A1w_ref, h0_ref, h1_ref, ss_ref, st_ref):
    h = jnp.dot(x_ref[...], W1_ref[...], preferred_element_type=jnp.float32)
    h = h + b1_ref[...][None, :]
    h = jnp.where(h > 0, h, jnp.exp(jnp.minimum(h, 0.0)) - 1.0)
    half = _D_HID // 2
    h0_ref[...] = h[:, :half]
    h1_ref[...] = h[:, half:]
    a = A1w_ref[...]
    ss_ref[...] = jnp.sum(h * a[:_D_HID, 0][None, :], axis=1)
    st_ref[...] = jnp.sum(h * a[_D_HID:, 0][None, :], axis=1)


@jax.jit
def _dense1(x, W1, b1, A1w):
    return pl.pallas_call(
        _dense1_body,
        out_shape=(
            jax.ShapeDtypeStruct((_N, _D_HID // 2), jnp.float32),
            jax.ShapeDtypeStruct((_N, _D_HID // 2), jnp.float32),
            jax.ShapeDtypeStruct((_N,), jnp.float32),
            jax.ShapeDtypeStruct((_N,), jnp.float32),
        ),
    )(x, W1, b1, A1w)


def _dense2_body(p0_ref, p1_ref, den_ref, W2_ref, b2_ref, A2w_ref,
                 h0_ref, h1_ref, ss_ref, st_ref):
    inv = 1.0 / (den_ref[...] + 1e-16)
    o0 = p0_ref[...] * inv[:, None]
    o1 = p1_ref[...] * inv[:, None]
    W2 = W2_ref[...]
    half_in = _D_HID // 2
    h = jnp.dot(o0, W2[:half_in], preferred_element_type=jnp.float32)
    h = h + jnp.dot(o1, W2[half_in:], preferred_element_type=jnp.float32)
    h = h + b2_ref[...][None, :]
    half = _D_OUT // 2
    h0_ref[...] = h[:, :half]
    h1_ref[...] = h[:, half:]
    a = A2w_ref[...]
    ss_ref[...] = jnp.sum(h * a[:_D_OUT, 0][None, :], axis=1)
    st_ref[...] = jnp.sum(h * a[_D_OUT:, 0][None, :], axis=1)


@jax.jit
def _dense2(p0, p1, den, W2, b2, A2w):
    return pl.pallas_call(
        _dense2_body,
        out_shape=(
            jax.ShapeDtypeStruct((_N, _D_OUT // 2), jnp.float32),
            jax.ShapeDtypeStruct((_N, _D_OUT // 2), jnp.float32),
            jax.ShapeDtypeStruct((_N,), jnp.float32),
            jax.ShapeDtypeStruct((_N,), jnp.float32),
        ),
    )(p0, p1, den, W2, b2, A2w)


def _final_body(p0_ref, p1_ref, den_ref, out_ref):
    inv = 1.0 / (den_ref[...] + 1e-16)
    o = jnp.concatenate([p0_ref[...], p1_ref[...]], axis=1) * inv[:, None]
    m = jnp.max(o, axis=1, keepdims=True)
    z = o - m
    lse = jnp.log(jnp.sum(jnp.exp(z), axis=1, keepdims=True))
    out_ref[...] = z - lse


@jax.jit
def _final(p0, p1, den):
    return pl.pallas_call(
        _final_body,
        out_shape=jax.ShapeDtypeStruct((_N, _D_OUT), jnp.float32),
    )(p0, p1, den)


def _aggregate_jnp(h0, h1, ss, st, Ab, src, tgt):
    logit = ss[src] + st[tgt] + Ab[0]
    logit = jnp.where(logit >= 0, logit, 0.2 * logit)
    w = jnp.exp(logit)
    den = jnp.zeros((_N,), jnp.float32).at[tgt].add(w)
    p0 = jnp.zeros_like(h0).at[tgt].add(h0[src] * w[:, None])
    p1 = jnp.zeros_like(h1).at[tgt].add(h1[src] * w[:, None])
    return p0, p1, den


def kernel(x, edge_index, W1, b1, A1w, A1b, W2, b2, A2w, A2b):
    src = edge_index[0]
    tgt = edge_index[1]
    h0, h1, ss, st = _dense1(x, W1, b1, A1w)
    p0, p1, den = _aggregate_jnp(h0, h1, ss, st, A1b, src, tgt)
    g0, g1, ss2, st2 = _dense2(p0, p1, den, W2, b2, A2w)
    q0, q1, den2 = _aggregate_jnp(g0, g1, ss2, st2, A2b, src, tgt)
    return _final(q0, q1, den2)


# speedup vs baseline: 12.5651x; 12.5651x over previous
import jax
import jax.numpy as jnp
from jax import lax
from jax.experimental import pallas as pl
from jax.experimental.pallas import tpu as pltpu
from jax.experimental.pallas import tpu_sc as plsc

_N = 10000
_N_PAD = 10240
_E = 320000
_D_IN = 128
_D_HID = 256
_D_OUT = 128
_HALF = 128

_NC = 2
_NS = 16
_CH = 80
_RPT = _N_PAD // _NS
_DROW = _N_PAD // _HALF


def _dense1_body(x_ref, W1_ref, b1_ref, A1w_ref, A1b_ref,
                 h0_ref, h1_ref, ss_ref, st_ref):
    h = jnp.dot(x_ref[...], W1_ref[...], preferred_element_type=jnp.float32)
    h = h + b1_ref[...][None, :]
    h = jnp.where(h > 0, h, jnp.exp(jnp.minimum(h, 0.0)) - 1.0)
    h0_ref[...] = h[:, :_HALF]
    h1_ref[...] = h[:, _HALF:]
    a = A1w_ref[...]
    ss_ref[...] = jnp.sum(h * a[:_D_HID, 0][None, :], axis=1) + A1b_ref[0]
    st_ref[...] = jnp.sum(h * a[_D_HID:, 0][None, :], axis=1)


@jax.jit
def _dense1(x, W1, b1, A1w, A1b):
    return pl.pallas_call(
        _dense1_body,
        out_shape=(
            jax.ShapeDtypeStruct((_N, _HALF), jnp.float32),
            jax.ShapeDtypeStruct((_N, _HALF), jnp.float32),
            jax.ShapeDtypeStruct((_N,), jnp.float32),
            jax.ShapeDtypeStruct((_N,), jnp.float32),
        ),
    )(x, W1, b1, A1w, A1b)


def _dense2_body(p0_ref, p1_ref, den_ref, W2_ref, b2_ref, A2w_ref, A2b_ref,
                 g_ref, ss_ref, st_ref):
    inv = 1.0 / (den_ref[...] + 1e-16)
    o0 = p0_ref[...] * inv[:, None]
    o1 = p1_ref[...] * inv[:, None]
    W2 = W2_ref[...]
    h = jnp.dot(o0, W2[:_HALF], preferred_element_type=jnp.float32)
    h = h + jnp.dot(o1, W2[_HALF:], preferred_element_type=jnp.float32)
    h = h + b2_ref[...][None, :]
    g_ref[...] = h
    a = A2w_ref[...]
    ss_ref[...] = jnp.sum(h * a[:_D_OUT, 0][None, :], axis=1) + A2b_ref[0]
    st_ref[...] = jnp.sum(h * a[_D_OUT:, 0][None, :], axis=1)


@jax.jit
def _dense2(p0, p1, den, W2, b2, A2w, A2b):
    return pl.pallas_call(
        _dense2_body,
        out_shape=(
            jax.ShapeDtypeStruct((_N, _D_OUT), jnp.float32),
            jax.ShapeDtypeStruct((_N,), jnp.float32),
            jax.ShapeDtypeStruct((_N,), jnp.float32),
        ),
    )(p0, p1, den, W2, b2, A2w, A2b)


def _final_body(q0_ref, q1_ref, den_ref, out_ref):
    inv = 1.0 / (den_ref[...] + 1e-16)
    o = (q0_ref[...] + q1_ref[...]) * inv[:, None]
    m = jnp.max(o, axis=1, keepdims=True)
    z = o - m
    lse = jnp.log(jnp.sum(jnp.exp(z), axis=1, keepdims=True))
    out_ref[...] = z - lse


@jax.jit
def _final(q0, q1, den):
    return pl.pallas_call(
        _final_body,
        out_shape=jax.ShapeDtypeStruct((_N, _D_OUT), jnp.float32),
    )(q0, q1, den)


_BLK = 5


def _make_agg(esplit):
    mesh = plsc.VectorSubcoreMesh(core_axis_name="c", subcore_axis_name="s")
    nch = (_E // (_NC * _NS) if esplit else _E // _NS) // _CH
    nblk = nch // _BLK

    def body(h0_hbm, h1_hbm, ss_hbm, st_hbm, src_hbm, tgt_hbm,
             p0_hbm, p1_hbm, den_hbm,
             ss_v, st_v, srcv, tgtv, rows, wv, denp, ridx, acc, dacc, sem):
        cid = lax.axis_index("c")
        sid = lax.axis_index("s")

        def zero_rows(r, carry):
            for k in range(_HALF // 16):
                rows[r, pl.ds(k * 16, 16)] = jnp.zeros((16,), jnp.float32)
                denp[r, pl.ds(k * 16, 16)] = jnp.zeros((16,), jnp.float32)
            return carry
        lax.fori_loop(0, _CH, zero_rows, 0)
        for g in range(_DROW // 16):
            ridx[pl.ds(g * 16, 16)] = lax.iota(jnp.int32, 16) + (g * 16)
        base_r = sid * _RPT
        for j in range(_RPT // _CH):
            pltpu.sync_copy(rows, acc.at[pl.ds(base_r + j * _CH, _CH)])
        @pl.when(sid < _DROW // 8)
        def _zden():
            pltpu.sync_copy(rows.at[pl.ds(0, 8)], dacc.at[pl.ds(sid * 8, 8)])

        pltpu.sync_copy(ss_hbm, ss_v)
        pltpu.sync_copy(st_hbm, st_v)
        wid = cid * _NS + sid if esplit else sid
        plsc.subcore_barrier()

        def edge_pass(h_hbm, do_den):
            def blk_body(b, carry):
                pltpu.sync_copy(src_hbm.at[wid].at[b], srcv)
                pltpu.sync_copy(tgt_hbm.at[wid].at[b], tgtv)
                for i in range(_BLK):
                    for g in range(_CH // 16):
                        si = srcv[i, pl.ds(g * 16, 16)]
                        ti = tgtv[i, pl.ds(g * 16, 16)]
                        logit = (plsc.load_gather(ss_v, [si])
                                 + plsc.load_gather(st_v, [ti]))
                        logit = jnp.where(logit >= 0.0, logit, 0.2 * logit)
                        w16 = jnp.exp(logit)
                        wv[pl.ds(g * 16, 16)] = w16
                        if do_den:
                            plsc.addupdate_scatter(
                                denp,
                                [lax.shift_right_logical(ti, 7),
                                 lax.bitwise_and(ti, 127)],
                                w16)
                    pltpu.async_copy(h_hbm.at[srcv.at[i]], rows, sem).wait()
                    def scale(e, c2):
                        we = wv[pl.ds(e, 16)][0]
                        for k in range(_HALF // 16):
                            rows[e, pl.ds(k * 16, 16)] = (
                                rows[e, pl.ds(k * 16, 16)] * we)
                        return c2
                    lax.fori_loop(0, _CH, scale, 0)
                    pltpu.sync_copy(rows, acc.at[tgtv.at[i]], add=True)
                return carry
            lax.fori_loop(0, nblk, blk_body, 0)
            if do_den:
                pltpu.sync_copy(denp, dacc.at[ridx], add=True)

        if esplit:
            edge_pass(h0_hbm, do_den=True)
        else:
            @pl.when(cid == 0)
            def _pass0():
                edge_pass(h0_hbm, do_den=True)

            @pl.when(cid == 1)
            def _pass1():
                edge_pass(h1_hbm, do_den=False)

        plsc.subcore_barrier()

        @pl.when(cid == 0)
        def _drain0():
            pltpu.sync_copy(acc.at[pl.ds(base_r, _RPT)],
                            p0_hbm.at[pl.ds(base_r, _RPT)])

        @pl.when(cid == 1)
        def _drain1():
            pltpu.sync_copy(acc.at[pl.ds(base_r, _RPT)],
                            p1_hbm.at[pl.ds(base_r, _RPT)])

        @pl.when(sid < _DROW // 8)
        def _draind():
            @pl.when(cid == 0)
            def _d0():
                pltpu.sync_copy(dacc.at[pl.ds(sid * 8, 8)],
                                den_hbm.at[0, pl.ds(sid * 8, 8)])
            if esplit:
                @pl.when(cid == 1)
                def _d1():
                    pltpu.sync_copy(dacc.at[pl.ds(sid * 8, 8)],
                                    den_hbm.at[1, pl.ds(sid * 8, 8)])

    return pl.kernel(
        body,
        out_type=(
            jax.ShapeDtypeStruct((_N_PAD, _HALF), jnp.float32),
            jax.ShapeDtypeStruct((_N_PAD, _HALF), jnp.float32),
            jax.ShapeDtypeStruct((2, _DROW, _HALF), jnp.float32),
        ),
        mesh=mesh,
        compiler_params=pltpu.CompilerParams(needs_layout_passes=False),
        scratch_types=[
            pltpu.VMEM((_N,), jnp.float32),
            pltpu.VMEM((_N,), jnp.float32),
            pltpu.VMEM((_BLK, _CH), jnp.int32),
            pltpu.VMEM((_BLK, _CH), jnp.int32),
            pltpu.VMEM((_CH, _HALF), jnp.float32),
            pltpu.VMEM((_CH + 16,), jnp.float32),
            pltpu.VMEM((_DROW, _HALF), jnp.float32),
            pltpu.VMEM((_DROW,), jnp.int32),
            pltpu.VMEM_SHARED((_N_PAD, _HALF), jnp.float32),
            pltpu.VMEM_SHARED((_DROW, _HALF), jnp.float32),
            pltpu.SemaphoreType.DMA,
        ],
    )


_agg1 = jax.jit(_make_agg(False))
_agg2 = jax.jit(_make_agg(True))


def kernel(x, edge_index, W1, b1, A1w, A1b, W2, b2, A2w, A2b):
    nblk1 = _E // _NS // _CH // _BLK
    nblk2 = _E // (_NC * _NS) // _CH // _BLK
    src1 = edge_index[0].reshape(_NS, nblk1, _BLK, _CH)
    tgt1 = edge_index[1].reshape(_NS, nblk1, _BLK, _CH)
    src2 = edge_index[0].reshape(_NC * _NS, nblk2, _BLK, _CH)
    tgt2 = edge_index[1].reshape(_NC * _NS, nblk2, _BLK, _CH)

    h0, h1, ss, st = _dense1(x, W1, b1, A1w, A1b)
    p0, p1, den1 = _agg1(h0, h1, ss, st, src1, tgt1)
    den1v = den1[0].reshape(_N_PAD)[:_N]
    g, ss2, st2 = _dense2(p0[:_N], p1[:_N], den1v, W2, b2, A2w, A2b)
    q0, q1, den2 = _agg2(g, g, ss2, st2, src2, tgt2)
    den2v = (den2[0] + den2[1]).reshape(_N_PAD)[:_N]
    return _final(q0[:_N], q1[:_N], den2v)


# speedup vs baseline: 17.3539x; 1.3811x over previous
import jax
import jax.numpy as jnp
from jax import lax
from jax.experimental import pallas as pl
from jax.experimental.pallas import tpu as pltpu
from jax.experimental.pallas import tpu_sc as plsc

_N = 10000
_N_PAD = 10240
_E = 320000
_D_IN = 128
_D_HID = 256
_D_OUT = 128
_HALF = 128

_NC = 2
_NS = 16
_CH = 80
_CHH = 40
_RPT = _N_PAD // _NS
_DROW = _N_PAD // _HALF


def _dense1_body(x_ref, W1_ref, b1_ref, A1w_ref, A1b_ref,
                 h0_ref, h1_ref, ss_ref, st_ref):
    h = jnp.dot(x_ref[...], W1_ref[...], preferred_element_type=jnp.float32)
    h = h + b1_ref[...][None, :]
    h = jnp.where(h > 0, h, jnp.exp(jnp.minimum(h, 0.0)) - 1.0)
    h0_ref[...] = h[:, :_HALF]
    h1_ref[...] = h[:, _HALF:]
    a = A1w_ref[...]
    ss_ref[...] = jnp.sum(h * a[:_D_HID, 0][None, :], axis=1) + A1b_ref[0]
    st_ref[...] = jnp.sum(h * a[_D_HID:, 0][None, :], axis=1)


@jax.jit
def _dense1(x, W1, b1, A1w, A1b):
    return pl.pallas_call(
        _dense1_body,
        out_shape=(
            jax.ShapeDtypeStruct((_N, _HALF), jnp.float32),
            jax.ShapeDtypeStruct((_N, _HALF), jnp.float32),
            jax.ShapeDtypeStruct((_N,), jnp.float32),
            jax.ShapeDtypeStruct((_N,), jnp.float32),
        ),
    )(x, W1, b1, A1w, A1b)


def _dense2_body(p0_ref, p1_ref, den_ref, W2_ref, b2_ref, A2w_ref, A2b_ref,
                 g_ref, ss_ref, st_ref):
    inv = 1.0 / (den_ref[...] + 1e-16)
    o0 = p0_ref[...] * inv[:, None]
    o1 = p1_ref[...] * inv[:, None]
    W2 = W2_ref[...]
    h = jnp.dot(o0, W2[:_HALF], preferred_element_type=jnp.float32)
    h = h + jnp.dot(o1, W2[_HALF:], preferred_element_type=jnp.float32)
    h = h + b2_ref[...][None, :]
    g_ref[...] = h
    a = A2w_ref[...]
    ss_ref[...] = jnp.sum(h * a[:_D_OUT, 0][None, :], axis=1) + A2b_ref[0]
    st_ref[...] = jnp.sum(h * a[_D_OUT:, 0][None, :], axis=1)


@jax.jit
def _dense2(p0, p1, den, W2, b2, A2w, A2b):
    return pl.pallas_call(
        _dense2_body,
        out_shape=(
            jax.ShapeDtypeStruct((_N, _D_OUT), jnp.float32),
            jax.ShapeDtypeStruct((_N,), jnp.float32),
            jax.ShapeDtypeStruct((_N,), jnp.float32),
        ),
    )(p0, p1, den, W2, b2, A2w, A2b)


def _final_body(q0_ref, q1_ref, den_ref, out_ref):
    inv = 1.0 / (den_ref[...] + 1e-16)
    o = (q0_ref[...] + q1_ref[...]) * inv[:, None]
    m = jnp.max(o, axis=1, keepdims=True)
    z = o - m
    lse = jnp.log(jnp.sum(jnp.exp(z), axis=1, keepdims=True))
    out_ref[...] = z - lse


@jax.jit
def _final(q0, q1, den):
    return pl.pallas_call(
        _final_body,
        out_shape=jax.ShapeDtypeStruct((_N, _D_OUT), jnp.float32),
    )(q0, q1, den)


_BLK = 25


def _make_agg(esplit):
    mesh = plsc.VectorSubcoreMesh(core_axis_name="c", subcore_axis_name="s")
    nch = (_E // (_NC * _NS) if esplit else _E // _NS) // _CH
    nblk = nch // _BLK

    def body(h0_hbm, h1_hbm, ss_hbm, st_hbm, src_hbm, tgt_hbm,
             p0_hbm, p1_hbm, den_hbm,
             ss_v, st_v, srcv, tgtv, rows0, rows1, wv,
             acc, dacc, sem0, sem1):
        cid = lax.axis_index("c")
        sid = lax.axis_index("s")
        rows = (rows0, rows1)
        sems = (sem0, sem1)

        def zero_rows(r, carry):
            for k in range(_HALF // 16):
                rows0[r, pl.ds(k * 16, 16)] = jnp.zeros((16,), jnp.float32)
            return carry
        lax.fori_loop(0, _CHH, zero_rows, 0)
        for k in range((_CH + 16) // 16):
            wv[pl.ds(k * 16, 16)] = jnp.zeros((16,), jnp.float32)
        base_r = sid * _RPT
        for j in range(_RPT // _CHH):
            pltpu.sync_copy(rows0, acc.at[pl.ds(base_r + j * _CHH, _CHH)])
        for j in range(_RPT // _CH):
            pltpu.sync_copy(wv.at[pl.ds(0, _CH)],
                            dacc.at[pl.ds(base_r + j * _CH, _CH)])

        pltpu.sync_copy(ss_hbm, ss_v)
        pltpu.sync_copy(st_hbm, st_v)
        wid = cid * _NS + sid if esplit else sid
        plsc.subcore_barrier()

        def edge_pass(h_hbm, do_den):
            def half_src(c, h):
                return h_hbm.at[srcv.at[c].at[pl.ds(h * _CHH, _CHH)]]

            def blk_body(b, carry):
                pltpu.sync_copy(src_hbm.at[wid].at[b], srcv)
                pltpu.sync_copy(tgt_hbm.at[wid].at[b], tgtv)
                pltpu.async_copy(half_src(0, 0), rows[0], sems[0])
                for i in range(_BLK):
                    for g in range(_CH // 16):
                        si = srcv[i, pl.ds(g * 16, 16)]
                        ti = tgtv[i, pl.ds(g * 16, 16)]
                        logit = (plsc.load_gather(ss_v, [si])
                                 + plsc.load_gather(st_v, [ti]))
                        logit = jnp.where(logit >= 0.0, logit, 0.2 * logit)
                        w16 = jnp.exp(logit)
                        wv[pl.ds(g * 16, 16)] = w16
                    if do_den:
                        pltpu.sync_copy(wv.at[pl.ds(0, _CH)],
                                        dacc.at[tgtv.at[i]], add=True)
                    for h in range(2):
                        g0 = 2 * i + h
                        nxt = g0 + 1
                        if nxt < 2 * _BLK:
                            ni, nh = divmod(nxt, 2)
                            pltpu.async_copy(half_src(ni, nh),
                                             rows[nxt % 2], sems[nxt % 2])
                        rb = rows[g0 % 2]
                        pltpu.make_async_copy(
                            half_src(i, h), rb, sems[g0 % 2]).wait()
                        def scale(e, c2):
                            we = wv[pl.ds(h * _CHH + e, 16)][0]
                            for k in range(_HALF // 16):
                                rb[e, pl.ds(k * 16, 16)] = (
                                    rb[e, pl.ds(k * 16, 16)] * we)
                            return c2
                        lax.fori_loop(0, _CHH, scale, 0)
                        pltpu.sync_copy(
                            rb,
                            acc.at[tgtv.at[i].at[pl.ds(h * _CHH, _CHH)]],
                            add=True)
                return carry
            lax.fori_loop(0, nblk, blk_body, 0)

        if esplit:
            edge_pass(h0_hbm, do_den=True)
        else:
            @pl.when(cid == 0)
            def _pass0():
                edge_pass(h0_hbm, do_den=True)

            @pl.when(cid == 1)
            def _pass1():
                edge_pass(h1_hbm, do_den=False)

        plsc.subcore_barrier()

        @pl.when(cid == 0)
        def _drain0():
            pltpu.sync_copy(acc.at[pl.ds(base_r, _RPT)],
                            p0_hbm.at[pl.ds(base_r, _RPT)])
            pltpu.sync_copy(dacc.at[pl.ds(base_r, _RPT)],
                            den_hbm.at[0].at[pl.ds(base_r, _RPT)])

        @pl.when(cid == 1)
        def _drain1():
            pltpu.sync_copy(acc.at[pl.ds(base_r, _RPT)],
                            p1_hbm.at[pl.ds(base_r, _RPT)])
            if esplit:
                pltpu.sync_copy(dacc.at[pl.ds(base_r, _RPT)],
                                den_hbm.at[1].at[pl.ds(base_r, _RPT)])

    return pl.kernel(
        body,
        out_type=(
            jax.ShapeDtypeStruct((_N_PAD, _HALF), jnp.float32),
            jax.ShapeDtypeStruct((_N_PAD, _HALF), jnp.float32),
            jax.ShapeDtypeStruct((2, _N_PAD), jnp.float32),
        ),
        mesh=mesh,
        compiler_params=pltpu.CompilerParams(needs_layout_passes=False),
        scratch_types=[
            pltpu.VMEM((_N,), jnp.float32),
            pltpu.VMEM((_N,), jnp.float32),
            pltpu.VMEM((_BLK, _CH), jnp.int32),
            pltpu.VMEM((_BLK, _CH), jnp.int32),
            pltpu.VMEM((_CHH, _HALF), jnp.float32),
            pltpu.VMEM((_CHH, _HALF), jnp.float32),
            pltpu.VMEM((_CH + 16,), jnp.float32),
            pltpu.VMEM_SHARED((_N_PAD, _HALF), jnp.float32),
            pltpu.VMEM_SHARED((_N_PAD,), jnp.float32),
            pltpu.SemaphoreType.DMA,
            pltpu.SemaphoreType.DMA,
        ],
    )


_agg1 = jax.jit(_make_agg(False))
_agg2 = jax.jit(_make_agg(True))


def kernel(x, edge_index, W1, b1, A1w, A1b, W2, b2, A2w, A2b):
    nblk1 = _E // _NS // _CH // _BLK
    nblk2 = _E // (_NC * _NS) // _CH // _BLK
    src1 = edge_index[0].reshape(_NS, nblk1, _BLK, _CH)
    tgt1 = edge_index[1].reshape(_NS, nblk1, _BLK, _CH)
    src2 = edge_index[0].reshape(_NC * _NS, nblk2, _BLK, _CH)
    tgt2 = edge_index[1].reshape(_NC * _NS, nblk2, _BLK, _CH)

    h0, h1, ss, st = _dense1(x, W1, b1, A1w, A1b)
    p0, p1, den1 = _agg1(h0, h1, ss, st, src1, tgt1)
    den1v = den1[0].reshape(_N_PAD)[:_N]
    g, ss2, st2 = _dense2(p0[:_N], p1[:_N], den1v, W2, b2, A2w, A2b)
    q0, q1, den2 = _agg2(g, g, ss2, st2, src2, tgt2)
    den2v = (den2[0] + den2[1]).reshape(_N_PAD)[:_N]
    return _final(q0[:_N], q1[:_N], den2v)


# speedup vs baseline: 18.4690x; 1.0643x over previous
import jax
import jax.numpy as jnp
from jax import lax
from jax.experimental import pallas as pl
from jax.experimental.pallas import tpu as pltpu
from jax.experimental.pallas import tpu_sc as plsc

_N = 10000
_N_PAD = 10240
_E = 320000
_D_IN = 128
_D_HID = 256
_D_OUT = 128
_HALF = 128

_NC = 2
_NS = 16
_CH = 80
_CHH = 40
_RPT = _N_PAD // _NS
_DROW = _N_PAD // _HALF


def _dense1_body(x_ref, W1_ref, b1_ref, A1w_ref, A1b_ref,
                 h0_ref, h1_ref, ss_ref, st_ref):
    h = jnp.dot(x_ref[...], W1_ref[...], preferred_element_type=jnp.float32)
    h = h + b1_ref[...][None, :]
    h = jnp.where(h > 0, h, jnp.exp(jnp.minimum(h, 0.0)) - 1.0)
    h0_ref[...] = h[:, :_HALF]
    h1_ref[...] = h[:, _HALF:]
    a = A1w_ref[...]
    ss_ref[...] = jnp.sum(h * a[:_D_HID, 0][None, :], axis=1) + A1b_ref[0]
    st_ref[...] = jnp.sum(h * a[_D_HID:, 0][None, :], axis=1)


@jax.jit
def _dense1(x, W1, b1, A1w, A1b):
    return pl.pallas_call(
        _dense1_body,
        out_shape=(
            jax.ShapeDtypeStruct((_N, _HALF), jnp.float32),
            jax.ShapeDtypeStruct((_N, _HALF), jnp.float32),
            jax.ShapeDtypeStruct((_N,), jnp.float32),
            jax.ShapeDtypeStruct((_N,), jnp.float32),
        ),
    )(x, W1, b1, A1w, A1b)


def _dense2_body(p0_ref, p1_ref, den_ref, W2_ref, b2_ref, A2w_ref, A2b_ref,
                 g_ref, ss_ref, st_ref):
    inv = 1.0 / (den_ref[...] + 1e-16)
    o0 = p0_ref[...] * inv[:, None]
    o1 = p1_ref[...] * inv[:, None]
    W2 = W2_ref[...]
    h = jnp.dot(o0, W2[:_HALF], preferred_element_type=jnp.float32)
    h = h + jnp.dot(o1, W2[_HALF:], preferred_element_type=jnp.float32)
    h = h + b2_ref[...][None, :]
    g_ref[...] = h
    a = A2w_ref[...]
    ss_ref[...] = jnp.sum(h * a[:_D_OUT, 0][None, :], axis=1) + A2b_ref[0]
    st_ref[...] = jnp.sum(h * a[_D_OUT:, 0][None, :], axis=1)


@jax.jit
def _dense2(p0, p1, den, W2, b2, A2w, A2b):
    return pl.pallas_call(
        _dense2_body,
        out_shape=(
            jax.ShapeDtypeStruct((_N, _D_OUT), jnp.float32),
            jax.ShapeDtypeStruct((_N,), jnp.float32),
            jax.ShapeDtypeStruct((_N,), jnp.float32),
        ),
    )(p0, p1, den, W2, b2, A2w, A2b)


def _final_body(q0_ref, q1_ref, den_ref, out_ref):
    inv = 1.0 / (den_ref[...] + 1e-16)
    o = (q0_ref[...] + q1_ref[...]) * inv[:, None]
    m = jnp.max(o, axis=1, keepdims=True)
    z = o - m
    lse = jnp.log(jnp.sum(jnp.exp(z), axis=1, keepdims=True))
    out_ref[...] = z - lse


@jax.jit
def _final(q0, q1, den):
    return pl.pallas_call(
        _final_body,
        out_shape=jax.ShapeDtypeStruct((_N, _D_OUT), jnp.float32),
    )(q0, q1, den)


_BLK = 25


def _make_agg(esplit):
    mesh = plsc.VectorSubcoreMesh(core_axis_name="c", subcore_axis_name="s")
    nch = (_E // (_NC * _NS) if esplit else _E // _NS) // _CH
    nblk = nch // _BLK

    def body(h0_hbm, h1_hbm, ss_hbm, st_hbm, src_hbm, tgt_hbm,
             p0_hbm, p1_hbm, den_hbm,
             ss_v, st_v, srcv, tgtv, rows0, rows1, wv,
             acc, dacc, sem0, sem1, ssem0, ssem1, dsem):
        cid = lax.axis_index("c")
        sid = lax.axis_index("s")
        rows = (rows0, rows1)
        sems = (sem0, sem1)
        ssems = (ssem0, ssem1)

        def zero_rows(r, carry):
            for k in range(_HALF // 16):
                rows0[r, pl.ds(k * 16, 16)] = jnp.zeros((16,), jnp.float32)
            return carry
        lax.fori_loop(0, _CHH, zero_rows, 0)
        for k in range((_CH + 16) // 16):
            wv[pl.ds(k * 16, 16)] = jnp.zeros((16,), jnp.float32)
        base_r = sid * _RPT
        for j in range(_RPT // _CHH):
            pltpu.sync_copy(rows0, acc.at[pl.ds(base_r + j * _CHH, _CHH)])
        for j in range(_RPT // _CH):
            pltpu.sync_copy(wv.at[pl.ds(0, _CH)],
                            dacc.at[pl.ds(base_r + j * _CH, _CH)])

        pltpu.sync_copy(ss_hbm, ss_v)
        pltpu.sync_copy(st_hbm, st_v)
        wid = cid * _NS + sid if esplit else sid
        plsc.subcore_barrier()

        def edge_pass(h_hbm, do_den):
            def half_src(c, h):
                return h_hbm.at[srcv.at[c].at[pl.ds(h * _CHH, _CHH)]]

            def acc_dst(c, h):
                return acc.at[tgtv.at[c].at[pl.ds(h * _CHH, _CHH)]]

            def blk_body(b, carry):
                pltpu.sync_copy(src_hbm.at[wid].at[b], srcv)
                pltpu.sync_copy(tgt_hbm.at[wid].at[b], tgtv)
                pltpu.async_copy(half_src(0, 0), rows[0], sems[0])
                for i in range(_BLK):
                    if do_den and i > 0:
                        pltpu.make_async_copy(
                            wv.at[pl.ds(0, _CH)],
                            dacc.at[tgtv.at[i - 1]], dsem).wait()
                    for g in range(_CH // 16):
                        si = srcv[i, pl.ds(g * 16, 16)]
                        ti = tgtv[i, pl.ds(g * 16, 16)]
                        logit = (plsc.load_gather(ss_v, [si])
                                 + plsc.load_gather(st_v, [ti]))
                        logit = jnp.where(logit >= 0.0, logit, 0.2 * logit)
                        w16 = jnp.exp(logit)
                        wv[pl.ds(g * 16, 16)] = w16
                    if do_den:
                        pltpu.async_copy(wv.at[pl.ds(0, _CH)],
                                         dacc.at[tgtv.at[i]], dsem, add=True)
                    for h in range(2):
                        g0 = 2 * i + h
                        nxt = g0 + 1
                        if nxt < 2 * _BLK:
                            ni, nh = divmod(nxt, 2)
                            if nxt >= 2:
                                pi, ph = divmod(nxt - 2, 2)
                                pltpu.make_async_copy(
                                    rows[nxt % 2], acc_dst(pi, ph),
                                    ssems[nxt % 2]).wait()
                            pltpu.async_copy(half_src(ni, nh),
                                             rows[nxt % 2], sems[nxt % 2])
                        rb = rows[g0 % 2]
                        pltpu.make_async_copy(
                            half_src(i, h), rb, sems[g0 % 2]).wait()
                        def scale(e, c2):
                            we = wv[pl.ds(h * _CHH + e, 16)][0]
                            for k in range(_HALF // 16):
                                rb[e, pl.ds(k * 16, 16)] = (
                                    rb[e, pl.ds(k * 16, 16)] * we)
                            return c2
                        lax.fori_loop(0, _CHH, scale, 0)
                        pltpu.async_copy(rb, acc_dst(i, h),
                                         ssems[g0 % 2], add=True)
                pltpu.make_async_copy(rows[0], acc_dst(_BLK - 1, 0),
                                      ssems[0]).wait()
                pltpu.make_async_copy(rows[1], acc_dst(_BLK - 1, 1),
                                      ssems[1]).wait()
                if do_den:
                    pltpu.make_async_copy(
                        wv.at[pl.ds(0, _CH)],
                        dacc.at[tgtv.at[_BLK - 1]], dsem).wait()
                return carry
            lax.fori_loop(0, nblk, blk_body, 0)

        if esplit:
            edge_pass(h0_hbm, do_den=True)
        else:
            @pl.when(cid == 0)
            def _pass0():
                edge_pass(h0_hbm, do_den=True)

            @pl.when(cid == 1)
            def _pass1():
                edge_pass(h1_hbm, do_den=False)

        plsc.subcore_barrier()

        @pl.when(cid == 0)
        def _drain0():
            pltpu.sync_copy(acc.at[pl.ds(base_r, _RPT)],
                            p0_hbm.at[pl.ds(base_r, _RPT)])
            pltpu.sync_copy(dacc.at[pl.ds(base_r, _RPT)],
                            den_hbm.at[0].at[pl.ds(base_r, _RPT)])

        @pl.when(cid == 1)
        def _drain1():
            pltpu.sync_copy(acc.at[pl.ds(base_r, _RPT)],
                            p1_hbm.at[pl.ds(base_r, _RPT)])
            if esplit:
                pltpu.sync_copy(dacc.at[pl.ds(base_r, _RPT)],
                                den_hbm.at[1].at[pl.ds(base_r, _RPT)])

    return pl.kernel(
        body,
        out_type=(
            jax.ShapeDtypeStruct((_N_PAD, _HALF), jnp.float32),
            jax.ShapeDtypeStruct((_N_PAD, _HALF), jnp.float32),
            jax.ShapeDtypeStruct((2, _N_PAD), jnp.float32),
        ),
        mesh=mesh,
        compiler_params=pltpu.CompilerParams(needs_layout_passes=False),
        scratch_types=[
            pltpu.VMEM((_N,), jnp.float32),
            pltpu.VMEM((_N,), jnp.float32),
            pltpu.VMEM((_BLK, _CH), jnp.int32),
            pltpu.VMEM((_BLK, _CH), jnp.int32),
            pltpu.VMEM((_CHH, _HALF), jnp.float32),
            pltpu.VMEM((_CHH, _HALF), jnp.float32),
            pltpu.VMEM((_CH + 16,), jnp.float32),
            pltpu.VMEM_SHARED((_N_PAD, _HALF), jnp.float32),
            pltpu.VMEM_SHARED((_N_PAD,), jnp.float32),
            pltpu.SemaphoreType.DMA,
            pltpu.SemaphoreType.DMA,
            pltpu.SemaphoreType.DMA,
            pltpu.SemaphoreType.DMA,
            pltpu.SemaphoreType.DMA,
        ],
    )


_agg1 = jax.jit(_make_agg(False))
_agg2 = jax.jit(_make_agg(True))


def kernel(x, edge_index, W1, b1, A1w, A1b, W2, b2, A2w, A2b):
    nblk1 = _E // _NS // _CH // _BLK
    nblk2 = _E // (_NC * _NS) // _CH // _BLK
    src1 = edge_index[0].reshape(_NS, nblk1, _BLK, _CH)
    tgt1 = edge_index[1].reshape(_NS, nblk1, _BLK, _CH)
    src2 = edge_index[0].reshape(_NC * _NS, nblk2, _BLK, _CH)
    tgt2 = edge_index[1].reshape(_NC * _NS, nblk2, _BLK, _CH)

    h0, h1, ss, st = _dense1(x, W1, b1, A1w, A1b)
    p0, p1, den1 = _agg1(h0, h1, ss, st, src1, tgt1)
    den1v = den1[0].reshape(_N_PAD)[:_N]
    g, ss2, st2 = _dense2(p0[:_N], p1[:_N], den1v, W2, b2, A2w, A2b)
    q0, q1, den2 = _agg2(g, g, ss2, st2, src2, tgt2)
    den2v = (den2[0] + den2[1]).reshape(_N_PAD)[:_N]
    return _final(q0[:_N], q1[:_N], den2v)


# speedup vs baseline: 20.1647x; 1.0918x over previous
import jax
import jax.numpy as jnp
from jax import lax
from jax.experimental import pallas as pl
from jax.experimental.pallas import tpu as pltpu
from jax.experimental.pallas import tpu_sc as plsc

_N = 10000
_N_PAD = 10240
_E = 320000
_D_IN = 128
_D_HID = 256
_D_OUT = 128
_HALF = 128

_NC = 2
_NS = 16
_CH = 80
_CHH = 40
_RPT = _N_PAD // _NS
_DROW = _N_PAD // _HALF


def _dense1_body(x_ref, W1_ref, b1_ref, A1w_ref, A1b_ref,
                 h0_ref, h1_ref, ss_ref, st_ref):
    h = jnp.dot(x_ref[...], W1_ref[...], preferred_element_type=jnp.float32)
    h = h + b1_ref[...][None, :]
    h = jnp.where(h > 0, h, jnp.exp(jnp.minimum(h, 0.0)) - 1.0)
    h0_ref[...] = h[:, :_HALF]
    h1_ref[...] = h[:, _HALF:]
    a = A1w_ref[...]
    ss_ref[...] = jnp.sum(h * a[:_D_HID, 0][None, :], axis=1) + A1b_ref[0]
    st_ref[...] = jnp.sum(h * a[_D_HID:, 0][None, :], axis=1)


@jax.jit
def _dense1(x, W1, b1, A1w, A1b):
    return pl.pallas_call(
        _dense1_body,
        out_shape=(
            jax.ShapeDtypeStruct((_N, _HALF), jnp.float32),
            jax.ShapeDtypeStruct((_N, _HALF), jnp.float32),
            jax.ShapeDtypeStruct((_N,), jnp.float32),
            jax.ShapeDtypeStruct((_N,), jnp.float32),
        ),
    )(x, W1, b1, A1w, A1b)


def _dense2_body(p0_ref, p1_ref, den_ref, W2_ref, b2_ref, A2w_ref, A2b_ref,
                 g_ref, ss_ref, st_ref):
    inv = 1.0 / (den_ref[...] + 1e-16)
    o0 = p0_ref[...] * inv[:, None]
    o1 = p1_ref[...] * inv[:, None]
    W2 = W2_ref[...]
    h = jnp.dot(o0, W2[:_HALF], preferred_element_type=jnp.float32)
    h = h + jnp.dot(o1, W2[_HALF:], preferred_element_type=jnp.float32)
    h = h + b2_ref[...][None, :]
    g_ref[...] = h
    a = A2w_ref[...]
    ss_ref[...] = jnp.sum(h * a[:_D_OUT, 0][None, :], axis=1) + A2b_ref[0]
    st_ref[...] = jnp.sum(h * a[_D_OUT:, 0][None, :], axis=1)


@jax.jit
def _dense2(p0, p1, den, W2, b2, A2w, A2b):
    return pl.pallas_call(
        _dense2_body,
        out_shape=(
            jax.ShapeDtypeStruct((_N, _D_OUT), jnp.float32),
            jax.ShapeDtypeStruct((_N,), jnp.float32),
            jax.ShapeDtypeStruct((_N,), jnp.float32),
        ),
    )(p0, p1, den, W2, b2, A2w, A2b)


def _final_body(q0_ref, q1_ref, den_ref, out_ref):
    inv = 1.0 / (den_ref[...] + 1e-16)
    o = (q0_ref[...] + q1_ref[...]) * inv[:, None]
    m = jnp.max(o, axis=1, keepdims=True)
    z = o - m
    lse = jnp.log(jnp.sum(jnp.exp(z), axis=1, keepdims=True))
    out_ref[...] = z - lse


@jax.jit
def _final(q0, q1, den):
    return pl.pallas_call(
        _final_body,
        out_shape=jax.ShapeDtypeStruct((_N, _D_OUT), jnp.float32),
    )(q0, q1, den)


_BLK = 25


def _make_agg(esplit):
    mesh = plsc.VectorSubcoreMesh(core_axis_name="c", subcore_axis_name="s")
    nch = (_E // (_NC * _NS) if esplit else _E // _NS) // _CH
    nblk = nch // _BLK

    def body(h0_hbm, h1_hbm, ss_hbm, st_hbm, src_hbm, tgt_hbm,
             p0_hbm, p1_hbm, den_hbm,
             ss_v, st_v, srcv, tgtv, rows0, rows1, wv,
             acc, dacc, sem0, sem1, ssem0, ssem1, dsem):
        cid = lax.axis_index("c")
        sid = lax.axis_index("s")
        rows = (rows0, rows1)
        sems = (sem0, sem1)
        ssems = (ssem0, ssem1)

        def zero_rows(r, carry):
            for k in range(_HALF // 16):
                rows0[r, pl.ds(k * 16, 16)] = jnp.zeros((16,), jnp.float32)
            return carry
        lax.fori_loop(0, _CHH, zero_rows, 0)
        for k in range((_CH + 16) // 16):
            wv[pl.ds(k * 16, 16)] = jnp.zeros((16,), jnp.float32)
        base_r = sid * _RPT
        for j in range(_RPT // _CHH):
            pltpu.sync_copy(rows0, acc.at[pl.ds(base_r + j * _CHH, _CHH)])
        for j in range(_RPT // _CH):
            pltpu.sync_copy(wv.at[pl.ds(0, _CH)],
                            dacc.at[pl.ds(base_r + j * _CH, _CH)])

        pltpu.sync_copy(ss_hbm, ss_v)
        pltpu.sync_copy(st_hbm, st_v)
        wid = cid * _NS + sid if esplit else sid
        plsc.subcore_barrier()

        def edge_pass(h_hbm, do_den):
            def half_src(c, h):
                return h_hbm.at[srcv.at[c].at[pl.ds(h * _CHH, _CHH)]]

            def acc_dst(c, h):
                return acc.at[tgtv.at[c].at[pl.ds(h * _CHH, _CHH)]]

            def blk_body(b, carry):
                pltpu.sync_copy(src_hbm.at[wid].at[b], srcv)
                pltpu.sync_copy(tgt_hbm.at[wid].at[b], tgtv)
                pltpu.async_copy(half_src(0, 0), rows[0], sems[0])

                def chunk_body(i, c):
                    @pl.when(i > 0)
                    def _w1():
                        pltpu.make_async_copy(rows[1], acc_dst(i - 1, 1),
                                              ssems[1]).wait()
                    pltpu.async_copy(half_src(i, 1), rows[1], sems[1])
                    if do_den:
                        @pl.when(i > 0)
                        def _wd():
                            pltpu.make_async_copy(
                                wv.at[pl.ds(0, _CH)],
                                dacc.at[tgtv.at[i - 1]], dsem).wait()
                    for g in range(_CH // 16):
                        si = srcv[i, pl.ds(g * 16, 16)]
                        ti = tgtv[i, pl.ds(g * 16, 16)]
                        logit = (plsc.load_gather(ss_v, [si])
                                 + plsc.load_gather(st_v, [ti]))
                        logit = jnp.where(logit >= 0.0, logit, 0.2 * logit)
                        w16 = jnp.exp(logit)
                        wv[pl.ds(g * 16, 16)] = w16
                    if do_den:
                        pltpu.async_copy(wv.at[pl.ds(0, _CH)],
                                         dacc.at[tgtv.at[i]], dsem, add=True)

                    def scale_half(h, rb):
                        def scale(j, c2):
                            e = j * 4
                            w4 = wv[pl.ds(h * _CHH + e, 16)]
                            for r in range(4):
                                we = w4[r]
                                for k in range(_HALF // 16):
                                    rb[e + r, pl.ds(k * 16, 16)] = (
                                        rb[e + r, pl.ds(k * 16, 16)] * we)
                            return c2
                        lax.fori_loop(0, _CHH // 4, scale, 0)

                    pltpu.make_async_copy(half_src(i, 0), rows[0],
                                          sems[0]).wait()
                    scale_half(0, rows[0])
                    pltpu.async_copy(rows[0], acc_dst(i, 0), ssems[0],
                                     add=True)
                    pltpu.make_async_copy(half_src(i, 1), rows[1],
                                          sems[1]).wait()
                    scale_half(1, rows[1])

                    @pl.when(i < _BLK - 1)
                    def _pref0():
                        pltpu.make_async_copy(rows[0], acc_dst(i, 0),
                                              ssems[0]).wait()
                        pltpu.async_copy(half_src(i + 1, 0), rows[0],
                                         sems[0])
                    pltpu.async_copy(rows[1], acc_dst(i, 1), ssems[1],
                                     add=True)
                    return c
                lax.fori_loop(0, _BLK, chunk_body, 0)
                pltpu.make_async_copy(rows[0], acc_dst(_BLK - 1, 0),
                                      ssems[0]).wait()
                pltpu.make_async_copy(rows[1], acc_dst(_BLK - 1, 1),
                                      ssems[1]).wait()
                if do_den:
                    pltpu.make_async_copy(
                        wv.at[pl.ds(0, _CH)],
                        dacc.at[tgtv.at[_BLK - 1]], dsem).wait()
                return carry
            lax.fori_loop(0, nblk, blk_body, 0)

        if esplit:
            edge_pass(h0_hbm, do_den=True)
        else:
            @pl.when(cid == 0)
            def _pass0():
                edge_pass(h0_hbm, do_den=True)

            @pl.when(cid == 1)
            def _pass1():
                edge_pass(h1_hbm, do_den=False)

        plsc.subcore_barrier()

        @pl.when(cid == 0)
        def _drain0():
            pltpu.sync_copy(acc.at[pl.ds(base_r, _RPT)],
                            p0_hbm.at[pl.ds(base_r, _RPT)])
            pltpu.sync_copy(dacc.at[pl.ds(base_r, _RPT)],
                            den_hbm.at[0].at[pl.ds(base_r, _RPT)])

        @pl.when(cid == 1)
        def _drain1():
            pltpu.sync_copy(acc.at[pl.ds(base_r, _RPT)],
                            p1_hbm.at[pl.ds(base_r, _RPT)])
            if esplit:
                pltpu.sync_copy(dacc.at[pl.ds(base_r, _RPT)],
                                den_hbm.at[1].at[pl.ds(base_r, _RPT)])

    return pl.kernel(
        body,
        out_type=(
            jax.ShapeDtypeStruct((_N_PAD, _HALF), jnp.float32),
            jax.ShapeDtypeStruct((_N_PAD, _HALF), jnp.float32),
            jax.ShapeDtypeStruct((2, _N_PAD), jnp.float32),
        ),
        mesh=mesh,
        compiler_params=pltpu.CompilerParams(needs_layout_passes=False),
        scratch_types=[
            pltpu.VMEM((_N,), jnp.float32),
            pltpu.VMEM((_N,), jnp.float32),
            pltpu.VMEM((_BLK, _CH), jnp.int32),
            pltpu.VMEM((_BLK, _CH), jnp.int32),
            pltpu.VMEM((_CHH, _HALF), jnp.float32),
            pltpu.VMEM((_CHH, _HALF), jnp.float32),
            pltpu.VMEM((_CH + 16,), jnp.float32),
            pltpu.VMEM_SHARED((_N_PAD, _HALF), jnp.float32),
            pltpu.VMEM_SHARED((_N_PAD,), jnp.float32),
            pltpu.SemaphoreType.DMA,
            pltpu.SemaphoreType.DMA,
            pltpu.SemaphoreType.DMA,
            pltpu.SemaphoreType.DMA,
            pltpu.SemaphoreType.DMA,
        ],
    )


_agg1 = jax.jit(_make_agg(False))
_agg2 = jax.jit(_make_agg(True))


def kernel(x, edge_index, W1, b1, A1w, A1b, W2, b2, A2w, A2b):
    nblk1 = _E // _NS // _CH // _BLK
    nblk2 = _E // (_NC * _NS) // _CH // _BLK
    src1 = edge_index[0].reshape(_NS, nblk1, _BLK, _CH)
    tgt1 = edge_index[1].reshape(_NS, nblk1, _BLK, _CH)
    src2 = edge_index[0].reshape(_NC * _NS, nblk2, _BLK, _CH)
    tgt2 = edge_index[1].reshape(_NC * _NS, nblk2, _BLK, _CH)

    h0, h1, ss, st = _dense1(x, W1, b1, A1w, A1b)
    p0, p1, den1 = _agg1(h0, h1, ss, st, src1, tgt1)
    den1v = den1[0].reshape(_N_PAD)[:_N]
    g, ss2, st2 = _dense2(p0[:_N], p1[:_N], den1v, W2, b2, A2w, A2b)
    q0, q1, den2 = _agg2(g, g, ss2, st2, src2, tgt2)
    den2v = (den2[0] + den2[1]).reshape(_N_PAD)[:_N]
    return _final(q0[:_N], q1[:_N], den2v)
